# baseline (speedup 1.0000x reference)
import functools

import jax
import jax.numpy as jnp
from jax import lax
from jax.experimental import pallas as pl
from jax.experimental.pallas import tpu as pltpu
from jax.experimental.pallas import tpu_sc as plsc

_NC = 2
_NS = 16
_NW = _NC * _NS
_L = 16
_CH = 112
_ROWW = 144

_NBK = 16
_NPB = 625
_DIVM = 6711
_DIVS = 22
_CAP16 = 112
_REG = 16 * _CAP16
_NCHMAX = 110
_CAPB = _NCHMAX * _CH

_SC_PARAMS = pltpu.CompilerParams(
    use_tc_tiling_on_sc=False, needs_layout_passes=False)
_MESH = dict(core_axis_name="c", subcore_axis_name="s")


def _mm_kernel(x_ref, wext_ref, wdst_ref, hext_ref, adst_ref):
    x = x_ref[...]
    hext_ref[...] = jnp.dot(x, wext_ref[...], preferred_element_type=jnp.float32)
    adst_ref[...] = jnp.dot(x, wdst_ref[...], preferred_element_type=jnp.float32)


def _ep_mm_kernel(acca_ref, accb_ref, rep_ref, b_ref, wext_ref, wdst_ref,
                  hext_ref, adst_ref):
    a = acca_ref[...] + accb_ref[...]
    den = jnp.dot(a[:, 128:144], rep_ref[...], preferred_element_type=jnp.float32)
    x = a[:, :128] / (den + 1e-16) + b_ref[...]
    x = jnp.where(x > 0, x, jnp.exp(x) - 1.0)
    hext_ref[...] = jnp.dot(x, wext_ref[...], preferred_element_type=jnp.float32)
    adst_ref[...] = jnp.dot(x, wdst_ref[...], preferred_element_type=jnp.float32)


def _ep_final_kernel(acca_ref, accb_ref, rep_ref, b_ref, out_ref):
    a = acca_ref[...] + accb_ref[...]
    den = jnp.dot(a[:, 128:144], rep_ref[...], preferred_element_type=jnp.float32)
    x = a[:, :128] / (den + 1e-16) + b_ref[...]
    out_ref[...] = jnp.where(x > 0, x, jnp.exp(x) - 1.0)


def _tc_project(x, wext, wdst, n_blocks=10):
    n = x.shape[0]
    blk = n // n_blocks
    d = x.shape[1]
    return pl.pallas_call(
        _mm_kernel,
        grid=(n_blocks,),
        in_specs=[
            pl.BlockSpec((blk, d), lambda i: (i, 0)),
            pl.BlockSpec((d, _ROWW), lambda i: (0, 0)),
            pl.BlockSpec((d, 16), lambda i: (0, 0)),
        ],
        out_specs=[
            pl.BlockSpec((blk, _ROWW), lambda i: (i, 0)),
            pl.BlockSpec((blk, 16), lambda i: (i, 0)),
        ],
        out_shape=[
            jax.ShapeDtypeStruct((n, _ROWW), jnp.float32),
            jax.ShapeDtypeStruct((n, 16), jnp.float32),
        ],
    )(x, wext, wdst)


def _tc_epilogue_project(acca, accb, rep, bias2d, wext, wdst, n_blocks=10):
    n = acca.shape[0]
    blk = n // n_blocks
    return pl.pallas_call(
        _ep_mm_kernel,
        grid=(n_blocks,),
        in_specs=[
            pl.BlockSpec((blk, _ROWW), lambda i: (i, 0)),
            pl.BlockSpec((blk, _ROWW), lambda i: (i, 0)),
            pl.BlockSpec((16, 128), lambda i: (0, 0)),
            pl.BlockSpec((1, 128), lambda i: (0, 0)),
            pl.BlockSpec((128, _ROWW), lambda i: (0, 0)),
            pl.BlockSpec((128, 16), lambda i: (0, 0)),
        ],
        out_specs=[
            pl.BlockSpec((blk, _ROWW), lambda i: (i, 0)),
            pl.BlockSpec((blk, 16), lambda i: (i, 0)),
        ],
        out_shape=[
            jax.ShapeDtypeStruct((n, _ROWW), jnp.float32),
            jax.ShapeDtypeStruct((n, 16), jnp.float32),
        ],
    )(acca, accb, rep, bias2d, wext, wdst)


def _tc_epilogue_final(acca, accb, rep, bias2d, n_blocks=10):
    n = acca.shape[0]
    blk = n // n_blocks
    return pl.pallas_call(
        _ep_final_kernel,
        grid=(n_blocks,),
        in_specs=[
            pl.BlockSpec((blk, _ROWW), lambda i: (i, 0)),
            pl.BlockSpec((blk, _ROWW), lambda i: (i, 0)),
            pl.BlockSpec((16, 128), lambda i: (0, 0)),
            pl.BlockSpec((1, 128), lambda i: (0, 0)),
        ],
        out_specs=pl.BlockSpec((blk, 128), lambda i: (i, 0)),
        out_shape=jax.ShapeDtypeStruct((n, 128), jnp.float32),
    )(acca, accb, rep, bias2d)


@jax.jit
def _sc_bucket_stage(src, dst):
    e = src.shape[0]
    ept = e // _NW
    assert ept % _L == 0

    @functools.partial(
        pl.kernel,
        out_type=[
            jax.ShapeDtypeStruct((_NW * _NBK * _REG,), jnp.int32),
            jax.ShapeDtypeStruct((_NW * _NBK * _REG,), jnp.int32),
            jax.ShapeDtypeStruct((_NW * _NBK * 16,), jnp.int32),
        ],
        mesh=plsc.VectorSubcoreMesh(**_MESH),
        compiler_params=_SC_PARAMS,
        scratch_types=[
            pltpu.VMEM((ept,), jnp.int32),
            pltpu.VMEM((ept,), jnp.int32),
            pltpu.VMEM((_NBK * _REG,), jnp.int32),
            pltpu.VMEM((_NBK * _REG,), jnp.int32),
            pltpu.VMEM((_NBK * 16,), jnp.int32),
        ],
    )
    def k(src_h, dst_h, ssrc_h, sdst_h, cnt_h, ebs, ebd, sts, std, cnt):
        cid = lax.axis_index("c")
        sid = lax.axis_index("s")
        wid = sid * _NC + cid
        ei = lax.iota(jnp.int32, _L)

        pltpu.sync_copy(src_h.at[pl.ds(wid * ept, ept)], ebs)
        pltpu.sync_copy(dst_h.at[pl.ds(wid * ept, ept)], ebd)

        def zc(i, _):
            cnt[pl.ds(i * _L, _L)] = jnp.zeros((_L,), jnp.int32)
            return 0
        lax.fori_loop(0, _NBK, zc, 0)

        def grp(g, _):
            sv = ebs[pl.ds(g * _L, _L)]
            dv = ebd[pl.ds(g * _L, _L)]
            bv = (dv * _DIVM) >> _DIVS
            cidx = bv * _L + ei
            c = plsc.load_gather(cnt, [cidx])
            plsc.store_scatter(cnt, [cidx], c + 1)
            slot = jnp.minimum(c, _CAP16 - 1)
            sidx = bv * _REG + slot * _L + ei
            plsc.store_scatter(sts, [sidx], sv)
            plsc.store_scatter(std, [sidx], dv)
            return 0
        lax.fori_loop(0, ept // _L, grp, 0)

        pltpu.sync_copy(sts, ssrc_h.at[pl.ds(wid * _NBK * _REG, _NBK * _REG)])
        pltpu.sync_copy(std, sdst_h.at[pl.ds(wid * _NBK * _REG, _NBK * _REG)])
        pltpu.sync_copy(cnt, cnt_h.at[pl.ds(wid * _NBK * 16, _NBK * 16)])

    return k(src, dst)


@functools.partial(jax.jit, static_argnames=("n_nodes",))
def _sc_compact(ssrc, sdst, counts, *, n_nodes):

    @functools.partial(
        pl.kernel,
        out_type=[
            jax.ShapeDtypeStruct((_NW * _CAPB,), jnp.int32),
            jax.ShapeDtypeStruct((_NW * _CAPB,), jnp.int32),
            jax.ShapeDtypeStruct((_NW * 16,), jnp.int32),
        ],
        mesh=plsc.VectorSubcoreMesh(**_MESH),
        compiler_params=_SC_PARAMS,
        scratch_types=[
            pltpu.VMEM((_NBK * _REG,), jnp.int32),
            pltpu.VMEM((_NBK * _REG,), jnp.int32),
            pltpu.VMEM((_NBK * 16,), jnp.int32),
            pltpu.VMEM((_CAPB,), jnp.int32),
            pltpu.VMEM((_CAPB,), jnp.int32),
            pltpu.VMEM((_L,), jnp.int32),
            pltpu.SemaphoreType.DMA,
        ],
    )
    def k(ssrc_h, sdst_h, cnt_h, dsrc_h, ddst_h, ntot_h,
          rs, rd, rc, ds_v, dd_v, nt_v, sem):
        cid = lax.axis_index("c")
        sid = lax.axis_index("s")
        wid = sid * _NC + cid
        ei = lax.iota(jnp.int32, _L)

        bw = wid // 2
        half = wid % 2
        cps = []
        for i in range(_NBK):
            t = half + 2 * i
            off = t * _NBK * _REG + bw * _REG
            cps.append(pltpu.async_copy(
                ssrc_h.at[pl.ds(off, _REG)], rs.at[pl.ds(i * _REG, _REG)], sem))
            cps.append(pltpu.async_copy(
                sdst_h.at[pl.ds(off, _REG)], rd.at[pl.ds(i * _REG, _REG)], sem))
            coff = t * _NBK * 16 + bw * 16
            cps.append(pltpu.async_copy(
                cnt_h.at[pl.ds(coff, 16)], rc.at[pl.ds(i * 16, 16)], sem))

        def zd(i, _):
            ds_v[pl.ds(i * _L, _L)] = jnp.zeros((_L,), jnp.int32)
            dd_v[pl.ds(i * _L, _L)] = jnp.zeros((_L,), jnp.int32)
            return 0
        lax.fori_loop(0, _CAPB // _L, zd, 0)
        for cp in cps:
            cp.wait()

        def region(t, cur):
            cvec = jnp.minimum(rc[pl.ds(t * 16, _L)], _CAP16)

            def slot(s, cur2):
                cur2 = jnp.minimum(cur2, _CAPB - _L)
                msk = cvec > s
                base = t * _REG + s * _L
                plsc.store_compressed(ds_v.at[pl.ds(cur2, _L)],
                                      rs[pl.ds(base, _L)], mask=msk)
                plsc.store_compressed(dd_v.at[pl.ds(cur2, _L)],
                                      rd[pl.ds(base, _L)], mask=msk)
                pc = plsc.all_reduce_population_count(msk)
                return cur2 + pc[0]
            return lax.fori_loop(0, _CAP16, slot, cur)
        cursor = lax.fori_loop(0, _NBK, region, jnp.int32(0))

        start = bw * _NPB + half * 313
        nb = 313 - half
        for s in range(20):
            lanes = s * _L + ei
            msk = lanes < nb
            vec = start + lanes
            cursor = jnp.minimum(cursor, _CAPB - _L)
            plsc.store_compressed(ds_v.at[pl.ds(cursor, _L)], vec, mask=msk)
            plsc.store_compressed(dd_v.at[pl.ds(cursor, _L)], vec, mask=msk)
            pc = plsc.all_reduce_population_count(msk)
            cursor = cursor + pc[0]

        nt_v[pl.ds(0, _L)] = jnp.broadcast_to(cursor, (_L,))
        pltpu.sync_copy(ds_v, dsrc_h.at[pl.ds(wid * _CAPB, _CAPB)])
        pltpu.sync_copy(dd_v, ddst_h.at[pl.ds(wid * _CAPB, _CAPB)])
        pltpu.sync_copy(nt_v, ntot_h.at[pl.ds(wid * _L, _L)])

    return k(ssrc, sdst, counts)


_NBUF = 2
_NIDX = 3
_UNROLL = 6


@functools.partial(jax.jit, static_argnames=("heads",))
def _sc_edge_pass(hext, adst_tab, dsrc, ddst, ntot, *, heads):
    out_ch = 128 // heads

    @functools.partial(
        pl.kernel,
        out_type=jax.ShapeDtypeStruct((_NW * _NPB * _ROWW,), jnp.float32),
        mesh=plsc.VectorSubcoreMesh(**_MESH),
        compiler_params=_SC_PARAMS,
        scratch_types=[
            [pltpu.VMEM((_CH,), jnp.int32)] * _NIDX,
            [pltpu.VMEM((_CH,), jnp.int32)] * _NIDX,
            [pltpu.VMEM((_CH, _ROWW), jnp.float32)] * _NBUF,
            [pltpu.VMEM((_CH, 16), jnp.float32)] * _NBUF,
            pltpu.VMEM((_NPB * _ROWW,), jnp.float32),
            pltpu.VMEM((_L,), jnp.int32),
            [pltpu.SemaphoreType.DMA] * _NBUF,
            [pltpu.SemaphoreType.DMA] * _NIDX,
        ],
    )
    def k(hext_h, adst_h, dsrc_h, ddst_h, ntot_h, out_h,
          srcidx, gdstidx, rows_v, adst_v, acc, nsm, gsem, isem):
        cid = lax.axis_index("c")
        sid = lax.axis_index("s")
        wid = sid * _NC + cid
        ei = lax.iota(jnp.int32, _L)
        ebase = wid * _CAPB

        pltpu.sync_copy(ntot_h.at[pl.ds(wid * _L, _L)], nsm)
        n_real = nsm[pl.ds(0, _L)][0]
        nch = (n_real + _CH - 1) // _CH

        nbase = (wid // 2) * _NPB

        def zacc(i, _):
            acc[pl.ds(i * _L, _L)] = jnp.zeros((_L,), jnp.float32)
            return 0
        lax.fori_loop(0, _NPB * _ROWW // _L, zacc, 0)

        def issue_idx(g, q):
            pltpu.async_copy(dsrc_h.at[pl.ds(ebase + g * _CH, _CH)],
                             srcidx[q], isem[q])
            pltpu.async_copy(ddst_h.at[pl.ds(ebase + g * _CH, _CH)],
                             gdstidx[q], isem[q])

        def wait_idx(q):
            pltpu.make_async_copy(dsrc_h.at[pl.ds(0, _CH)], srcidx[q], isem[q]).wait()
            pltpu.make_async_copy(dsrc_h.at[pl.ds(0, _CH)], gdstidx[q], isem[q]).wait()

        def issue_gather(b, q):
            pltpu.async_copy(hext_h.at[srcidx[q]], rows_v[b], gsem[b])
            pltpu.async_copy(adst_h.at[gdstidx[q]], adst_v[b], gsem[b])

        def wait_gather(b):
            pltpu.make_async_copy(hext_h.at[pl.ds(0, _CH)], rows_v[b], gsem[b]).wait()
            pltpu.make_async_copy(adst_h.at[pl.ds(0, _CH)], adst_v[b], gsem[b]).wait()

        def compute(g, b, q):
            base = g * _CH
            rv = rows_v[b]
            av = adst_v[b]
            gq = gdstidx[q]

            def escale(p, _):
                ws = []
                idxs = []
                for ee in range(4):
                    e = 4 * p + ee
                    efull = jnp.full((_L,), e, jnp.int32)
                    a_s = rv[e, pl.ds(128, _L)]
                    a_d = av[e, pl.ds(0, _L)]
                    t = a_s + a_d
                    t = jnp.where(t >= 0, t, 0.2 * t)
                    valid = (base + e) < n_real
                    w16 = jnp.where((ei < heads) & valid, jnp.exp(t), 0.0)
                    ldb = plsc.load_gather(gq, [efull]) - nbase
                    ldb = jnp.minimum(jnp.maximum(ldb, 0), _NPB - 1)
                    ws.append(w16)
                    idxs.append(ldb * _ROWW + ei)
                for j in range(8):
                    hj = (j * 16) // out_ch
                    hjf = jnp.full((_L,), hj, jnp.int32)
                    for ee in range(4):
                        e = 4 * p + ee
                        wb = ws[ee].at[hjf].get(mode="promise_in_bounds")
                        plsc.addupdate_scatter(
                            acc, [idxs[ee] + j * 16],
                            rv[e, pl.ds(j * 16, 16)] * wb)
                for ee in range(4):
                    plsc.addupdate_scatter(acc, [idxs[ee] + 128], ws[ee])
                return 0
            lax.fori_loop(0, _CH // 4, escale, 0)

        issue_idx(0, 0)
        issue_idx(1, 1)
        wait_idx(0)
        issue_gather(0, 0)

        def trip(t, _):
            for kk in range(_UNROLL):
                g = _UNROLL * t + kk
                b = kk % _NBUF
                bn = (kk + 1) % _NBUF
                qn = (kk + 1) % _NIDX
                qnn = (kk + 2) % _NIDX

                @pl.when(g < nch)
                def _():
                    @pl.when(g + 1 < nch)
                    def _():
                        wait_idx(qn)
                        issue_gather(bn, qn)

                    @pl.when(g + 2 < nch)
                    def _():
                        issue_idx(g + 2, qnn)
                    wait_gather(b)
                    compute(g, b, kk % _NIDX)
            return 0
        lax.fori_loop(0, (nch + _UNROLL - 1) // _UNROLL, trip, 0)

        pltpu.sync_copy(acc, out_h.at[pl.ds(wid * _NPB * _ROWW, _NPB * _ROWW)])

    return k(hext, adst_tab, dsrc, ddst, ntot)


def _fold_weights(W, att_src, att_dst, heads, out_ch):
    w3 = W.reshape(W.shape[0], heads, out_ch)
    wsrc = jnp.sum(w3 * att_src, axis=-1)
    wdst = jnp.sum(w3 * att_dst, axis=-1)
    pad = jnp.zeros((W.shape[0], 16 - heads), jnp.float32)
    wext = jnp.concatenate([W, wsrc, pad], axis=1)
    wdst16 = jnp.concatenate([wdst, pad], axis=1)
    return wext, wdst16


def _rep_matrix(heads):
    out_ch = 128 // heads
    rep = jnp.zeros((16, 128), jnp.float32)
    hc = jnp.arange(128) // out_ch
    rep = rep.at[hc, jnp.arange(128)].set(1.0)
    return rep


def kernel(inputs, edge_index, W1, att_src1, att_dst1, bias1,
           W2, att_src2, att_dst2, bias2):
    N, D = inputs.shape
    E = edge_index.shape[1]
    assert E % (_NW * _L) == 0 and _NW * _NPB >= N

    wext1, wdst1 = _fold_weights(W1, att_src1, att_dst1, 8, 16)
    wext2, wdst2 = _fold_weights(W2, att_src2, att_dst2, 1, 128)
    rep1 = _rep_matrix(8)
    rep2 = _rep_matrix(1)
    b1 = bias1.reshape(1, 128)
    b2 = bias2.reshape(1, 128)

    ssrc, sdst, counts = _sc_bucket_stage(edge_index[0], edge_index[1])
    dsrc, ddst, ntot = _sc_compact(ssrc, sdst, counts, n_nodes=N)

    def halves(flat):
        o = flat.reshape(_NBK, 2, _NPB, _ROWW)
        return (o[:, 0].reshape(N, _ROWW), o[:, 1].reshape(N, _ROWW))

    hext1, adst1 = _tc_project(inputs, wext1, wdst1)
    a1, b1h = halves(_sc_edge_pass(hext1, adst1, dsrc, ddst, ntot, heads=8))
    hext2, adst2 = _tc_epilogue_project(a1, b1h, rep1, b1, wext2, wdst2)
    a2, b2h = halves(_sc_edge_pass(hext2, adst2, dsrc, ddst, ntot, heads=1))
    return _tc_epilogue_final(a2, b2h, rep2, b2)


# speedup vs baseline: 1.1180x; 1.1180x over previous
import functools

import jax
import jax.numpy as jnp
from jax import lax
from jax.experimental import pallas as pl
from jax.experimental.pallas import tpu as pltpu
from jax.experimental.pallas import tpu_sc as plsc

_NC = 2
_NS = 16
_NW = _NC * _NS
_L = 16
_CH = 112
_ROWW = 144

_NPB = 313
_DIVM = 214406
_DIVS = 26
_CAP16 = 56
_REG = 16 * _CAP16
_NCHMAX = 110
_CAPB = _NCHMAX * _CH

_SC_PARAMS = pltpu.CompilerParams(
    use_tc_tiling_on_sc=False, needs_layout_passes=False)
_MESH = dict(core_axis_name="c", subcore_axis_name="s")


def _mm_kernel(x_ref, wext_ref, wdst_ref, hext_ref, adst_ref):
    x = x_ref[...]
    hext_ref[...] = jnp.dot(x, wext_ref[...], preferred_element_type=jnp.float32)
    adst_ref[...] = jnp.dot(x, wdst_ref[...], preferred_element_type=jnp.float32)


def _ep_mm_kernel(acc_ref, rep_ref, b_ref, wext_ref, wdst_ref,
                  hext_ref, adst_ref):
    a = acc_ref[...]
    den = jnp.dot(a[:, 128:144], rep_ref[...], preferred_element_type=jnp.float32)
    x = a[:, :128] / (den + 1e-16) + b_ref[...]
    x = jnp.where(x > 0, x, jnp.exp(x) - 1.0)
    hext_ref[...] = jnp.dot(x, wext_ref[...], preferred_element_type=jnp.float32)
    adst_ref[...] = jnp.dot(x, wdst_ref[...], preferred_element_type=jnp.float32)


def _ep_final_kernel(acc_ref, rep_ref, b_ref, out_ref):
    a = acc_ref[...]
    den = jnp.dot(a[:, 128:144], rep_ref[...], preferred_element_type=jnp.float32)
    x = a[:, :128] / (den + 1e-16) + b_ref[...]
    out_ref[...] = jnp.where(x > 0, x, jnp.exp(x) - 1.0)


def _tc_project(x, wext, wdst, n_blocks=10):
    n = x.shape[0]
    blk = n // n_blocks
    d = x.shape[1]
    return pl.pallas_call(
        _mm_kernel,
        grid=(n_blocks,),
        in_specs=[
            pl.BlockSpec((blk, d), lambda i: (i, 0)),
            pl.BlockSpec((d, _ROWW), lambda i: (0, 0)),
            pl.BlockSpec((d, 16), lambda i: (0, 0)),
        ],
        out_specs=[
            pl.BlockSpec((blk, _ROWW), lambda i: (i, 0)),
            pl.BlockSpec((blk, 16), lambda i: (i, 0)),
        ],
        out_shape=[
            jax.ShapeDtypeStruct((n, _ROWW), jnp.float32),
            jax.ShapeDtypeStruct((n, 16), jnp.float32),
        ],
    )(x, wext, wdst)


def _tc_epilogue_project(acc, rep, bias2d, wext, wdst, n_blocks=10):
    n = acc.shape[0]
    blk = n // n_blocks
    return pl.pallas_call(
        _ep_mm_kernel,
        grid=(n_blocks,),
        in_specs=[
            pl.BlockSpec((blk, _ROWW), lambda i: (i, 0)),
            pl.BlockSpec((16, 128), lambda i: (0, 0)),
            pl.BlockSpec((1, 128), lambda i: (0, 0)),
            pl.BlockSpec((128, _ROWW), lambda i: (0, 0)),
            pl.BlockSpec((128, 16), lambda i: (0, 0)),
        ],
        out_specs=[
            pl.BlockSpec((blk, _ROWW), lambda i: (i, 0)),
            pl.BlockSpec((blk, 16), lambda i: (i, 0)),
        ],
        out_shape=[
            jax.ShapeDtypeStruct((n, _ROWW), jnp.float32),
            jax.ShapeDtypeStruct((n, 16), jnp.float32),
        ],
    )(acc, rep, bias2d, wext, wdst)


def _tc_epilogue_final(acc, rep, bias2d, n_blocks=10):
    n = acc.shape[0]
    blk = n // n_blocks
    return pl.pallas_call(
        _ep_final_kernel,
        grid=(n_blocks,),
        in_specs=[
            pl.BlockSpec((blk, _ROWW), lambda i: (i, 0)),
            pl.BlockSpec((16, 128), lambda i: (0, 0)),
            pl.BlockSpec((1, 128), lambda i: (0, 0)),
        ],
        out_specs=pl.BlockSpec((blk, 128), lambda i: (i, 0)),
        out_shape=jax.ShapeDtypeStruct((n, 128), jnp.float32),
    )(acc, rep, bias2d)


@jax.jit
def _sc_bucket_stage(src, dst):
    e = src.shape[0]
    ept = e // _NW
    assert ept % _L == 0

    @functools.partial(
        pl.kernel,
        out_type=[
            jax.ShapeDtypeStruct((_NW * _NW * _REG,), jnp.int32),
            jax.ShapeDtypeStruct((_NW * _NW * _REG,), jnp.int32),
            jax.ShapeDtypeStruct((_NW * _NW * 16,), jnp.int32),
        ],
        mesh=plsc.VectorSubcoreMesh(**_MESH),
        compiler_params=_SC_PARAMS,
        scratch_types=[
            pltpu.VMEM((ept,), jnp.int32),
            pltpu.VMEM((ept,), jnp.int32),
            pltpu.VMEM((_NW * _REG,), jnp.int32),
            pltpu.VMEM((_NW * _REG,), jnp.int32),
            pltpu.VMEM((_NW * 16,), jnp.int32),
        ],
    )
    def k(src_h, dst_h, ssrc_h, sdst_h, cnt_h, ebs, ebd, sts, std, cnt):
        cid = lax.axis_index("c")
        sid = lax.axis_index("s")
        wid = sid * _NC + cid
        ei = lax.iota(jnp.int32, _L)

        pltpu.sync_copy(src_h.at[pl.ds(wid * ept, ept)], ebs)
        pltpu.sync_copy(dst_h.at[pl.ds(wid * ept, ept)], ebd)

        def zc(i, _):
            cnt[pl.ds(i * _L, _L)] = jnp.zeros((_L,), jnp.int32)
            return 0
        lax.fori_loop(0, _NW, zc, 0)

        def grp(g, _):
            sv = ebs[pl.ds(g * _L, _L)]
            dv = ebd[pl.ds(g * _L, _L)]
            bv = (dv * _DIVM) >> _DIVS
            cidx = bv * _L + ei
            c = plsc.load_gather(cnt, [cidx])
            plsc.store_scatter(cnt, [cidx], c + 1)
            slot = jnp.minimum(c, _CAP16 - 1)
            sidx = bv * _REG + slot * _L + ei
            plsc.store_scatter(sts, [sidx], sv)
            plsc.store_scatter(std, [sidx], dv)
            return 0
        lax.fori_loop(0, ept // _L, grp, 0)

        pltpu.sync_copy(sts, ssrc_h.at[pl.ds(wid * _NW * _REG, _NW * _REG)])
        pltpu.sync_copy(std, sdst_h.at[pl.ds(wid * _NW * _REG, _NW * _REG)])
        pltpu.sync_copy(cnt, cnt_h.at[pl.ds(wid * _NW * 16, _NW * 16)])

    return k(src, dst)


@functools.partial(jax.jit, static_argnames=("n_nodes",))
def _sc_compact(ssrc, sdst, counts, *, n_nodes):

    @functools.partial(
        pl.kernel,
        out_type=[
            jax.ShapeDtypeStruct((_NW * _CAPB,), jnp.int32),
            jax.ShapeDtypeStruct((_NW * _CAPB,), jnp.int32),
            jax.ShapeDtypeStruct((_NW * 16,), jnp.int32),
        ],
        mesh=plsc.VectorSubcoreMesh(**_MESH),
        compiler_params=_SC_PARAMS,
        scratch_types=[
            pltpu.VMEM((_NW * _REG,), jnp.int32),
            pltpu.VMEM((_NW * _REG,), jnp.int32),
            pltpu.VMEM((_NW * 16,), jnp.int32),
            pltpu.VMEM((_CAPB,), jnp.int32),
            pltpu.VMEM((_CAPB,), jnp.int32),
            pltpu.VMEM((_L,), jnp.int32),
            pltpu.SemaphoreType.DMA,
        ],
    )
    def k(ssrc_h, sdst_h, cnt_h, dsrc_h, ddst_h, ntot_h,
          rs, rd, rc, ds_v, dd_v, nt_v, sem):
        cid = lax.axis_index("c")
        sid = lax.axis_index("s")
        wid = sid * _NC + cid
        ei = lax.iota(jnp.int32, _L)

        cps = []
        for t in range(_NW):
            off = t * _NW * _REG + wid * _REG
            cps.append(pltpu.async_copy(
                ssrc_h.at[pl.ds(off, _REG)], rs.at[pl.ds(t * _REG, _REG)], sem))
            cps.append(pltpu.async_copy(
                sdst_h.at[pl.ds(off, _REG)], rd.at[pl.ds(t * _REG, _REG)], sem))
            coff = t * _NW * 16 + wid * 16
            cps.append(pltpu.async_copy(
                cnt_h.at[pl.ds(coff, 16)], rc.at[pl.ds(t * 16, 16)], sem))

        def zd(i, _):
            ds_v[pl.ds(i * _L, _L)] = jnp.zeros((_L,), jnp.int32)
            dd_v[pl.ds(i * _L, _L)] = jnp.zeros((_L,), jnp.int32)
            return 0
        lax.fori_loop(0, _CAPB // _L, zd, 0)
        for cp in cps:
            cp.wait()

        def region(t, cur):
            cvec = jnp.minimum(rc[pl.ds(t * 16, _L)], _CAP16)

            def slot(s, cur2):
                cur2 = jnp.minimum(cur2, _CAPB - _L)
                msk = cvec > s
                base = t * _REG + s * _L
                plsc.store_compressed(ds_v.at[pl.ds(cur2, _L)],
                                      rs[pl.ds(base, _L)], mask=msk)
                plsc.store_compressed(dd_v.at[pl.ds(cur2, _L)],
                                      rd[pl.ds(base, _L)], mask=msk)
                pc = plsc.all_reduce_population_count(msk)
                return cur2 + pc[0]
            return lax.fori_loop(0, _CAP16, slot, cur)
        cursor = lax.fori_loop(0, _NW, region, jnp.int32(0))

        nb = jnp.minimum(n_nodes - wid * _NPB, _NPB)
        for s in range((_NPB + _L - 1) // _L):
            lanes = s * _L + ei
            msk = lanes < nb
            vec = wid * _NPB + lanes
            cursor = jnp.minimum(cursor, _CAPB - _L)
            plsc.store_compressed(ds_v.at[pl.ds(cursor, _L)], vec, mask=msk)
            plsc.store_compressed(dd_v.at[pl.ds(cursor, _L)], vec, mask=msk)
            pc = plsc.all_reduce_population_count(msk)
            cursor = cursor + pc[0]

        nt_v[pl.ds(0, _L)] = jnp.broadcast_to(cursor, (_L,))
        pltpu.sync_copy(ds_v, dsrc_h.at[pl.ds(wid * _CAPB, _CAPB)])
        pltpu.sync_copy(dd_v, ddst_h.at[pl.ds(wid * _CAPB, _CAPB)])
        pltpu.sync_copy(nt_v, ntot_h.at[pl.ds(wid * _L, _L)])

    return k(ssrc, sdst, counts)


_NBUF = 2
_NIDX = 3
_UNROLL = 6


@functools.partial(jax.jit, static_argnames=("heads",))
def _sc_edge_pass(hext, adst_tab, dsrc, ddst, ntot, *, heads):
    out_ch = 128 // heads

    @functools.partial(
        pl.kernel,
        out_type=jax.ShapeDtypeStruct((_NW * _NPB * _ROWW,), jnp.float32),
        mesh=plsc.VectorSubcoreMesh(**_MESH),
        compiler_params=_SC_PARAMS,
        scratch_types=[
            [pltpu.VMEM((_CH,), jnp.int32)] * _NIDX,
            [pltpu.VMEM((_CH,), jnp.int32)] * _NIDX,
            [pltpu.VMEM((_CH, _ROWW), jnp.float32)] * _NBUF,
            [pltpu.VMEM((_CH, 16), jnp.float32)] * _NBUF,
            pltpu.VMEM((_NPB * _ROWW,), jnp.float32),
            pltpu.VMEM((_L,), jnp.int32),
            [pltpu.SemaphoreType.DMA] * _NBUF,
            [pltpu.SemaphoreType.DMA] * _NIDX,
        ],
    )
    def k(hext_h, adst_h, dsrc_h, ddst_h, ntot_h, out_h,
          srcidx, gdstidx, rows_v, adst_v, acc, nsm, gsem, isem):
        cid = lax.axis_index("c")
        sid = lax.axis_index("s")
        wid = sid * _NC + cid
        ei = lax.iota(jnp.int32, _L)
        ebase = wid * _CAPB

        pltpu.sync_copy(ntot_h.at[pl.ds(wid * _L, _L)], nsm)
        n_real = nsm[pl.ds(0, _L)][0]
        nch = (n_real + _CH - 1) // _CH

        def zacc(i, _):
            acc[pl.ds(i * _L, _L)] = jnp.zeros((_L,), jnp.float32)
            return 0
        lax.fori_loop(0, _NPB * _ROWW // _L, zacc, 0)

        def issue_idx(g, q):
            pltpu.async_copy(dsrc_h.at[pl.ds(ebase + g * _CH, _CH)],
                             srcidx[q], isem[q])
            pltpu.async_copy(ddst_h.at[pl.ds(ebase + g * _CH, _CH)],
                             gdstidx[q], isem[q])

        def wait_idx(q):
            pltpu.make_async_copy(dsrc_h.at[pl.ds(0, _CH)], srcidx[q], isem[q]).wait()
            pltpu.make_async_copy(dsrc_h.at[pl.ds(0, _CH)], gdstidx[q], isem[q]).wait()

        def issue_gather(b, q):
            pltpu.async_copy(hext_h.at[srcidx[q]], rows_v[b], gsem[b])
            pltpu.async_copy(adst_h.at[gdstidx[q]], adst_v[b], gsem[b])

        def wait_gather(b):
            pltpu.make_async_copy(hext_h.at[pl.ds(0, _CH)], rows_v[b], gsem[b]).wait()
            pltpu.make_async_copy(adst_h.at[pl.ds(0, _CH)], adst_v[b], gsem[b]).wait()

        def compute(g, b, q):
            base = g * _CH
            rv = rows_v[b]
            av = adst_v[b]
            gq = gdstidx[q]

            def escale(p, _):
                ws = []
                idxs = []
                for ee in range(4):
                    e = 4 * p + ee
                    efull = jnp.full((_L,), e, jnp.int32)
                    a_s = rv[e, pl.ds(128, _L)]
                    a_d = av[e, pl.ds(0, _L)]
                    t = a_s + a_d
                    t = jnp.where(t >= 0, t, 0.2 * t)
                    valid = (base + e) < n_real
                    w16 = jnp.where((ei < heads) & valid, jnp.exp(t), 0.0)
                    ldb = plsc.load_gather(gq, [efull]) - wid * _NPB
                    ldb = jnp.minimum(jnp.maximum(ldb, 0), _NPB - 1)
                    ws.append(w16)
                    idxs.append(ldb * _ROWW + ei)
                for j in range(8):
                    hj = (j * 16) // out_ch
                    hjf = jnp.full((_L,), hj, jnp.int32)
                    for ee in range(4):
                        e = 4 * p + ee
                        wb = ws[ee].at[hjf].get(mode="promise_in_bounds")
                        plsc.addupdate_scatter(
                            acc, [idxs[ee] + j * 16],
                            rv[e, pl.ds(j * 16, 16)] * wb)
                for ee in range(4):
                    plsc.addupdate_scatter(acc, [idxs[ee] + 128], ws[ee])
                return 0
            lax.fori_loop(0, _CH // 4, escale, 0)

        issue_idx(0, 0)
        issue_idx(1, 1)
        wait_idx(0)
        issue_gather(0, 0)

        def trip(t, _):
            for kk in range(_UNROLL):
                g = _UNROLL * t + kk
                b = kk % _NBUF
                bn = (kk + 1) % _NBUF
                qn = (kk + 1) % _NIDX
                qnn = (kk + 2) % _NIDX

                @pl.when(g < nch)
                def _():
                    @pl.when(g + 1 < nch)
                    def _():
                        wait_idx(qn)
                        issue_gather(bn, qn)

                    @pl.when(g + 2 < nch)
                    def _():
                        issue_idx(g + 2, qnn)
                    wait_gather(b)
                    compute(g, b, kk % _NIDX)
            return 0
        lax.fori_loop(0, (nch + _UNROLL - 1) // _UNROLL, trip, 0)

        pltpu.sync_copy(acc, out_h.at[pl.ds(wid * _NPB * _ROWW, _NPB * _ROWW)])

    return k(hext, adst_tab, dsrc, ddst, ntot).reshape(_NW * _NPB, _ROWW)


def _fold_weights(W, att_src, att_dst, heads, out_ch):
    w3 = W.reshape(W.shape[0], heads, out_ch)
    wsrc = jnp.sum(w3 * att_src, axis=-1)
    wdst = jnp.sum(w3 * att_dst, axis=-1)
    pad = jnp.zeros((W.shape[0], 16 - heads), jnp.float32)
    wext = jnp.concatenate([W, wsrc, pad], axis=1)
    wdst16 = jnp.concatenate([wdst, pad], axis=1)
    return wext, wdst16


def _rep_matrix(heads):
    out_ch = 128 // heads
    rep = jnp.zeros((16, 128), jnp.float32)
    hc = jnp.arange(128) // out_ch
    rep = rep.at[hc, jnp.arange(128)].set(1.0)
    return rep


def kernel(inputs, edge_index, W1, att_src1, att_dst1, bias1,
           W2, att_src2, att_dst2, bias2):
    N, D = inputs.shape
    E = edge_index.shape[1]
    assert E % (_NW * _L) == 0 and _NW * _NPB >= N

    wext1, wdst1 = _fold_weights(W1, att_src1, att_dst1, 8, 16)
    wext2, wdst2 = _fold_weights(W2, att_src2, att_dst2, 1, 128)
    rep1 = _rep_matrix(8)
    rep2 = _rep_matrix(1)
    b1 = bias1.reshape(1, 128)
    b2 = bias2.reshape(1, 128)

    ssrc, sdst, counts = _sc_bucket_stage(edge_index[0], edge_index[1])
    dsrc, ddst, ntot = _sc_compact(ssrc, sdst, counts, n_nodes=N)

    hext1, adst1 = _tc_project(inputs, wext1, wdst1)
    acc1 = _sc_edge_pass(hext1, adst1, dsrc, ddst, ntot, heads=8)
    hext2, adst2 = _tc_epilogue_project(acc1[:N], rep1, b1, wext2, wdst2)
    acc2 = _sc_edge_pass(hext2, adst2, dsrc, ddst, ntot, heads=1)
    return _tc_epilogue_final(acc2[:N], rep2, b2)


# speedup vs baseline: 1.1471x; 1.0260x over previous
import functools

import jax
import jax.numpy as jnp
from jax import lax
from jax.experimental import pallas as pl
from jax.experimental.pallas import tpu as pltpu
from jax.experimental.pallas import tpu_sc as plsc

_NC = 2
_NS = 16
_NW = _NC * _NS
_L = 16
_CH = 112
_ROWW = 144

_NPB = 313
_DIVM = 214406
_DIVS = 26
_CAP16 = 56
_REG = 16 * _CAP16
_NCHMAX = 110
_CAPB = _NCHMAX * _CH

_SC_PARAMS = pltpu.CompilerParams(
    use_tc_tiling_on_sc=False, needs_layout_passes=False)
_MESH = dict(core_axis_name="c", subcore_axis_name="s")


def _mm_kernel(x_ref, wext_ref, wdst_ref, hext_ref, adst_ref):
    x = x_ref[...]
    hext_ref[...] = jnp.dot(x, wext_ref[...], preferred_element_type=jnp.float32)
    adst_ref[...] = jnp.dot(x, wdst_ref[...], preferred_element_type=jnp.float32)


def _ep_mm_kernel(acc_ref, rep_ref, b_ref, wext_ref, wdst_ref,
                  hext_ref, adst_ref):
    a = acc_ref[...]
    den = jnp.dot(a[:, 128:144], rep_ref[...], preferred_element_type=jnp.float32)
    x = a[:, :128] / (den + 1e-16) + b_ref[...]
    x = jnp.where(x > 0, x, jnp.exp(x) - 1.0)
    hext_ref[...] = jnp.dot(x, wext_ref[...], preferred_element_type=jnp.float32)
    adst_ref[...] = jnp.dot(x, wdst_ref[...], preferred_element_type=jnp.float32)


def _ep_final_kernel(acc_ref, rep_ref, b_ref, out_ref):
    a = acc_ref[...]
    den = jnp.dot(a[:, 128:144], rep_ref[...], preferred_element_type=jnp.float32)
    x = a[:, :128] / (den + 1e-16) + b_ref[...]
    out_ref[...] = jnp.where(x > 0, x, jnp.exp(x) - 1.0)


def _tc_project(x, wext, wdst, n_blocks=10):
    n = x.shape[0]
    blk = n // n_blocks
    d = x.shape[1]
    return pl.pallas_call(
        _mm_kernel,
        grid=(n_blocks,),
        in_specs=[
            pl.BlockSpec((blk, d), lambda i: (i, 0)),
            pl.BlockSpec((d, _ROWW), lambda i: (0, 0)),
            pl.BlockSpec((d, 16), lambda i: (0, 0)),
        ],
        out_specs=[
            pl.BlockSpec((blk, _ROWW), lambda i: (i, 0)),
            pl.BlockSpec((blk, 16), lambda i: (i, 0)),
        ],
        out_shape=[
            jax.ShapeDtypeStruct((n, _ROWW), jnp.float32),
            jax.ShapeDtypeStruct((n, 16), jnp.float32),
        ],
    )(x, wext, wdst)


def _tc_epilogue_project(acc, rep, bias2d, wext, wdst, n_blocks=10):
    n = acc.shape[0]
    blk = n // n_blocks
    return pl.pallas_call(
        _ep_mm_kernel,
        grid=(n_blocks,),
        in_specs=[
            pl.BlockSpec((blk, _ROWW), lambda i: (i, 0)),
            pl.BlockSpec((16, 128), lambda i: (0, 0)),
            pl.BlockSpec((1, 128), lambda i: (0, 0)),
            pl.BlockSpec((128, _ROWW), lambda i: (0, 0)),
            pl.BlockSpec((128, 16), lambda i: (0, 0)),
        ],
        out_specs=[
            pl.BlockSpec((blk, _ROWW), lambda i: (i, 0)),
            pl.BlockSpec((blk, 16), lambda i: (i, 0)),
        ],
        out_shape=[
            jax.ShapeDtypeStruct((n, _ROWW), jnp.float32),
            jax.ShapeDtypeStruct((n, 16), jnp.float32),
        ],
    )(acc, rep, bias2d, wext, wdst)


def _tc_epilogue_final(acc, rep, bias2d, n_blocks=10):
    n = acc.shape[0]
    blk = n // n_blocks
    return pl.pallas_call(
        _ep_final_kernel,
        grid=(n_blocks,),
        in_specs=[
            pl.BlockSpec((blk, _ROWW), lambda i: (i, 0)),
            pl.BlockSpec((16, 128), lambda i: (0, 0)),
            pl.BlockSpec((1, 128), lambda i: (0, 0)),
        ],
        out_specs=pl.BlockSpec((blk, 128), lambda i: (i, 0)),
        out_shape=jax.ShapeDtypeStruct((n, 128), jnp.float32),
    )(acc, rep, bias2d)


@jax.jit
def _sc_bucket_stage(src, dst):
    e = src.shape[0]
    ept = e // _NW
    assert ept % _L == 0

    @functools.partial(
        pl.kernel,
        out_type=[
            jax.ShapeDtypeStruct((_NW * _NW * _REG,), jnp.int32),
            jax.ShapeDtypeStruct((_NW * _NW * _REG,), jnp.int32),
            jax.ShapeDtypeStruct((_NW * _NW * 16,), jnp.int32),
        ],
        mesh=plsc.VectorSubcoreMesh(**_MESH),
        compiler_params=_SC_PARAMS,
        scratch_types=[
            pltpu.VMEM((ept,), jnp.int32),
            pltpu.VMEM((ept,), jnp.int32),
            pltpu.VMEM((_NW * _REG,), jnp.int32),
            pltpu.VMEM((_NW * _REG,), jnp.int32),
            pltpu.VMEM((_NW * 16,), jnp.int32),
        ],
    )
    def k(src_h, dst_h, ssrc_h, sdst_h, cnt_h, ebs, ebd, sts, std, cnt):
        cid = lax.axis_index("c")
        sid = lax.axis_index("s")
        wid = sid * _NC + cid
        ei = lax.iota(jnp.int32, _L)

        pltpu.sync_copy(src_h.at[pl.ds(wid * ept, ept)], ebs)
        pltpu.sync_copy(dst_h.at[pl.ds(wid * ept, ept)], ebd)

        def zc(i, _):
            cnt[pl.ds(i * _L, _L)] = jnp.zeros((_L,), jnp.int32)
            return 0
        lax.fori_loop(0, _NW, zc, 0)

        def grp(g, _):
            sv = ebs[pl.ds(g * _L, _L)]
            dv = ebd[pl.ds(g * _L, _L)]
            bv = (dv * _DIVM) >> _DIVS
            cidx = bv * _L + ei
            c = plsc.load_gather(cnt, [cidx])
            plsc.store_scatter(cnt, [cidx], c + 1)
            slot = jnp.minimum(c, _CAP16 - 1)
            sidx = bv * _REG + slot * _L + ei
            plsc.store_scatter(sts, [sidx], sv)
            plsc.store_scatter(std, [sidx], dv)
            return 0
        lax.fori_loop(0, ept // _L, grp, 0)

        pltpu.sync_copy(sts, ssrc_h.at[pl.ds(wid * _NW * _REG, _NW * _REG)])
        pltpu.sync_copy(std, sdst_h.at[pl.ds(wid * _NW * _REG, _NW * _REG)])
        pltpu.sync_copy(cnt, cnt_h.at[pl.ds(wid * _NW * 16, _NW * 16)])

    return k(src, dst)


@functools.partial(jax.jit, static_argnames=("n_nodes",))
def _sc_compact(ssrc, sdst, counts, *, n_nodes):

    @functools.partial(
        pl.kernel,
        out_type=[
            jax.ShapeDtypeStruct((_NW * _CAPB,), jnp.int32),
            jax.ShapeDtypeStruct((_NW * _CAPB,), jnp.int32),
            jax.ShapeDtypeStruct((_NW * 16,), jnp.int32),
        ],
        mesh=plsc.VectorSubcoreMesh(**_MESH),
        compiler_params=_SC_PARAMS,
        scratch_types=[
            pltpu.VMEM((_NW * _REG,), jnp.int32),
            pltpu.VMEM((_NW * _REG,), jnp.int32),
            pltpu.VMEM((_NW * 16,), jnp.int32),
            pltpu.VMEM((_CAPB,), jnp.int32),
            pltpu.VMEM((_CAPB,), jnp.int32),
            pltpu.VMEM((_L,), jnp.int32),
            pltpu.SemaphoreType.DMA,
        ],
    )
    def k(ssrc_h, sdst_h, cnt_h, dsrc_h, ddst_h, ntot_h,
          rs, rd, rc, ds_v, dd_v, nt_v, sem):
        cid = lax.axis_index("c")
        sid = lax.axis_index("s")
        wid = sid * _NC + cid
        ei = lax.iota(jnp.int32, _L)

        cps = []
        for t in range(_NW):
            off = t * _NW * _REG + wid * _REG
            cps.append(pltpu.async_copy(
                ssrc_h.at[pl.ds(off, _REG)], rs.at[pl.ds(t * _REG, _REG)], sem))
            cps.append(pltpu.async_copy(
                sdst_h.at[pl.ds(off, _REG)], rd.at[pl.ds(t * _REG, _REG)], sem))
            coff = t * _NW * 16 + wid * 16
            cps.append(pltpu.async_copy(
                cnt_h.at[pl.ds(coff, 16)], rc.at[pl.ds(t * 16, 16)], sem))

        def zd(i, _):
            ds_v[pl.ds(i * _L, _L)] = jnp.zeros((_L,), jnp.int32)
            dd_v[pl.ds(i * _L, _L)] = jnp.zeros((_L,), jnp.int32)
            return 0
        lax.fori_loop(0, _CAPB // _L, zd, 0)
        for cp in cps:
            cp.wait()

        def region(t, cur):
            cvec = jnp.minimum(rc[pl.ds(t * 16, _L)], _CAP16)

            def slot(s, cur2):
                cur2 = jnp.minimum(cur2, _CAPB - _L)
                msk = cvec > s
                base = t * _REG + s * _L
                plsc.store_compressed(ds_v.at[pl.ds(cur2, _L)],
                                      rs[pl.ds(base, _L)], mask=msk)
                plsc.store_compressed(dd_v.at[pl.ds(cur2, _L)],
                                      rd[pl.ds(base, _L)], mask=msk)
                pc = plsc.all_reduce_population_count(msk)
                return cur2 + pc[0]
            return lax.fori_loop(0, _CAP16, slot, cur)
        cursor = lax.fori_loop(0, _NW, region, jnp.int32(0))

        nb = jnp.minimum(n_nodes - wid * _NPB, _NPB)
        for s in range((_NPB + _L - 1) // _L):
            lanes = s * _L + ei
            msk = lanes < nb
            vec = wid * _NPB + lanes
            cursor = jnp.minimum(cursor, _CAPB - _L)
            plsc.store_compressed(ds_v.at[pl.ds(cursor, _L)], vec, mask=msk)
            plsc.store_compressed(dd_v.at[pl.ds(cursor, _L)], vec, mask=msk)
            pc = plsc.all_reduce_population_count(msk)
            cursor = cursor + pc[0]

        nt_v[pl.ds(0, _L)] = jnp.broadcast_to(cursor, (_L,))
        pltpu.sync_copy(ds_v, dsrc_h.at[pl.ds(wid * _CAPB, _CAPB)])
        pltpu.sync_copy(dd_v, ddst_h.at[pl.ds(wid * _CAPB, _CAPB)])
        pltpu.sync_copy(nt_v, ntot_h.at[pl.ds(wid * _L, _L)])

    return k(ssrc, sdst, counts)


_NBUF = 2
_NIDX = 3
_UNROLL = 6


@functools.partial(jax.jit, static_argnames=("heads",))
def _sc_edge_pass(hext, adst_tab, dsrc, ddst, ntot, *, heads):
    out_ch = 128 // heads

    @functools.partial(
        pl.kernel,
        out_type=jax.ShapeDtypeStruct((_NW * _NPB * _ROWW,), jnp.float32),
        mesh=plsc.VectorSubcoreMesh(**_MESH),
        compiler_params=_SC_PARAMS,
        scratch_types=[
            [pltpu.VMEM((_CH,), jnp.int32)] * _NIDX,
            [pltpu.VMEM((_CH,), jnp.int32)] * _NIDX,
            [pltpu.VMEM((_CH, _ROWW), jnp.float32)] * _NBUF,
            [pltpu.VMEM((_CH, 16), jnp.float32)] * _NBUF,
            pltpu.VMEM((_NPB * _ROWW,), jnp.float32),
            pltpu.VMEM((_L,), jnp.int32),
            [pltpu.SemaphoreType.DMA] * _NBUF,
            [pltpu.SemaphoreType.DMA] * _NIDX,
        ],
    )
    def k(hext_h, adst_h, dsrc_h, ddst_h, ntot_h, out_h,
          srcidx, gdstidx, rows_v, adst_v, acc, nsm, gsem, isem):
        cid = lax.axis_index("c")
        sid = lax.axis_index("s")
        wid = sid * _NC + cid
        ei = lax.iota(jnp.int32, _L)
        ebase = wid * _CAPB

        pltpu.sync_copy(ntot_h.at[pl.ds(wid * _L, _L)], nsm)
        n_real = nsm[pl.ds(0, _L)][0]
        nch = (n_real + _CH - 1) // _CH

        def zacc(i, _):
            acc[pl.ds(i * _L, _L)] = jnp.zeros((_L,), jnp.float32)
            return 0
        lax.fori_loop(0, _NPB * _ROWW // _L, zacc, 0)

        def issue_idx(g, q):
            pltpu.async_copy(dsrc_h.at[pl.ds(ebase + g * _CH, _CH)],
                             srcidx[q], isem[q])
            pltpu.async_copy(ddst_h.at[pl.ds(ebase + g * _CH, _CH)],
                             gdstidx[q], isem[q])

        def wait_idx(q):
            pltpu.make_async_copy(dsrc_h.at[pl.ds(0, _CH)], srcidx[q], isem[q]).wait()
            pltpu.make_async_copy(dsrc_h.at[pl.ds(0, _CH)], gdstidx[q], isem[q]).wait()

        def issue_gather(b, q):
            pltpu.async_copy(hext_h.at[srcidx[q]], rows_v[b], gsem[b])
            pltpu.async_copy(adst_h.at[gdstidx[q]], adst_v[b], gsem[b])

        def wait_gather(b):
            pltpu.make_async_copy(hext_h.at[pl.ds(0, _CH)], rows_v[b], gsem[b]).wait()
            pltpu.make_async_copy(adst_h.at[pl.ds(0, _CH)], adst_v[b], gsem[b]).wait()

        def compute(g, b, q):
            base = g * _CH
            rv = rows_v[b]
            av = adst_v[b]
            gq = gdstidx[q]

            def escale(p, _):
                ws = []
                idxs = []
                for ee in range(8):
                    e = 8 * p + ee
                    efull = jnp.full((_L,), e, jnp.int32)
                    a_s = rv[e, pl.ds(128, _L)]
                    a_d = av[e, pl.ds(0, _L)]
                    t = a_s + a_d
                    t = jnp.where(t >= 0, t, 0.2 * t)
                    valid = (base + e) < n_real
                    w16 = jnp.where((ei < heads) & valid, jnp.exp(t), 0.0)
                    ldb = plsc.load_gather(gq, [efull]) - wid * _NPB
                    ldb = jnp.minimum(jnp.maximum(ldb, 0), _NPB - 1)
                    ws.append(w16)
                    idxs.append(ldb * _ROWW + ei)
                for j in range(8):
                    hj = (j * 16) // out_ch
                    hjf = jnp.full((_L,), hj, jnp.int32)
                    for ee in range(8):
                        e = 8 * p + ee
                        wb = ws[ee].at[hjf].get(mode="promise_in_bounds")
                        plsc.addupdate_scatter(
                            acc, [idxs[ee] + j * 16],
                            rv[e, pl.ds(j * 16, 16)] * wb)
                for ee in range(8):
                    plsc.addupdate_scatter(acc, [idxs[ee] + 128], ws[ee])
                return 0
            lax.fori_loop(0, _CH // 8, escale, 0)

        issue_idx(0, 0)
        issue_idx(1, 1)
        wait_idx(0)
        issue_gather(0, 0)

        def trip(t, _):
            for kk in range(_UNROLL):
                g = _UNROLL * t + kk
                b = kk % _NBUF
                bn = (kk + 1) % _NBUF
                qn = (kk + 1) % _NIDX
                qnn = (kk + 2) % _NIDX

                @pl.when(g < nch)
                def _():
                    @pl.when(g + 1 < nch)
                    def _():
                        wait_idx(qn)
                        issue_gather(bn, qn)

                    @pl.when(g + 2 < nch)
                    def _():
                        issue_idx(g + 2, qnn)
                    wait_gather(b)
                    compute(g, b, kk % _NIDX)
            return 0
        lax.fori_loop(0, (nch + _UNROLL - 1) // _UNROLL, trip, 0)

        pltpu.sync_copy(acc, out_h.at[pl.ds(wid * _NPB * _ROWW, _NPB * _ROWW)])

    return k(hext, adst_tab, dsrc, ddst, ntot).reshape(_NW * _NPB, _ROWW)


def _fold_weights(W, att_src, att_dst, heads, out_ch):
    w3 = W.reshape(W.shape[0], heads, out_ch)
    wsrc = jnp.sum(w3 * att_src, axis=-1)
    wdst = jnp.sum(w3 * att_dst, axis=-1)
    pad = jnp.zeros((W.shape[0], 16 - heads), jnp.float32)
    wext = jnp.concatenate([W, wsrc, pad], axis=1)
    wdst16 = jnp.concatenate([wdst, pad], axis=1)
    return wext, wdst16


def _rep_matrix(heads):
    out_ch = 128 // heads
    rep = jnp.zeros((16, 128), jnp.float32)
    hc = jnp.arange(128) // out_ch
    rep = rep.at[hc, jnp.arange(128)].set(1.0)
    return rep


def kernel(inputs, edge_index, W1, att_src1, att_dst1, bias1,
           W2, att_src2, att_dst2, bias2):
    N, D = inputs.shape
    E = edge_index.shape[1]
    assert E % (_NW * _L) == 0 and _NW * _NPB >= N

    wext1, wdst1 = _fold_weights(W1, att_src1, att_dst1, 8, 16)
    wext2, wdst2 = _fold_weights(W2, att_src2, att_dst2, 1, 128)
    rep1 = _rep_matrix(8)
    rep2 = _rep_matrix(1)
    b1 = bias1.reshape(1, 128)
    b2 = bias2.reshape(1, 128)

    ssrc, sdst, counts = _sc_bucket_stage(edge_index[0], edge_index[1])
    dsrc, ddst, ntot = _sc_compact(ssrc, sdst, counts, n_nodes=N)

    hext1, adst1 = _tc_project(inputs, wext1, wdst1)
    acc1 = _sc_edge_pass(hext1, adst1, dsrc, ddst, ntot, heads=8)
    hext2, adst2 = _tc_epilogue_project(acc1[:N], rep1, b1, wext2, wdst2)
    acc2 = _sc_edge_pass(hext2, adst2, dsrc, ddst, ntot, heads=1)
    return _tc_epilogue_final(acc2[:N], rep2, b2)


# speedup vs baseline: 1.1472x; 1.0001x over previous
import functools

import jax
import jax.numpy as jnp
from jax import lax
from jax.experimental import pallas as pl
from jax.experimental.pallas import tpu as pltpu
from jax.experimental.pallas import tpu_sc as plsc

_NC = 2
_NS = 16
_NW = _NC * _NS
_L = 16
_CH = 112
_ROWW = 144

_NPB = 313
_DIVM = 214406
_DIVS = 26
_CAP16 = 56
_REG = 16 * _CAP16
_NCHMAX = 110
_CAPB = _NCHMAX * _CH

_SC_PARAMS = pltpu.CompilerParams(
    use_tc_tiling_on_sc=False, needs_layout_passes=False)
_MESH = dict(core_axis_name="c", subcore_axis_name="s")


def _mm_kernel(x_ref, wext_ref, wdst_ref, hext_ref, adst_ref):
    x = x_ref[...]
    hext_ref[...] = jnp.dot(x, wext_ref[...], preferred_element_type=jnp.float32)
    adst_ref[...] = jnp.dot(x, wdst_ref[...], preferred_element_type=jnp.float32)


def _ep_mm_kernel(acc_ref, rep_ref, b_ref, wext_ref, wdst_ref,
                  hext_ref, adst_ref):
    a = acc_ref[...]
    den = jnp.dot(a[:, 128:144], rep_ref[...], preferred_element_type=jnp.float32)
    x = a[:, :128] / (den + 1e-16) + b_ref[...]
    x = jnp.where(x > 0, x, jnp.exp(x) - 1.0)
    hext_ref[...] = jnp.dot(x, wext_ref[...], preferred_element_type=jnp.float32)
    adst_ref[...] = jnp.dot(x, wdst_ref[...], preferred_element_type=jnp.float32)


def _ep_final_kernel(acc_ref, rep_ref, b_ref, out_ref):
    a = acc_ref[...]
    den = jnp.dot(a[:, 128:144], rep_ref[...], preferred_element_type=jnp.float32)
    x = a[:, :128] / (den + 1e-16) + b_ref[...]
    out_ref[...] = jnp.where(x > 0, x, jnp.exp(x) - 1.0)


def _tc_project(x, wext, wdst, n_blocks=10):
    n = x.shape[0]
    blk = n // n_blocks
    d = x.shape[1]
    return pl.pallas_call(
        _mm_kernel,
        grid=(n_blocks,),
        in_specs=[
            pl.BlockSpec((blk, d), lambda i: (i, 0)),
            pl.BlockSpec((d, _ROWW), lambda i: (0, 0)),
            pl.BlockSpec((d, 16), lambda i: (0, 0)),
        ],
        out_specs=[
            pl.BlockSpec((blk, _ROWW), lambda i: (i, 0)),
            pl.BlockSpec((blk, 16), lambda i: (i, 0)),
        ],
        out_shape=[
            jax.ShapeDtypeStruct((n, _ROWW), jnp.float32),
            jax.ShapeDtypeStruct((n, 16), jnp.float32),
        ],
    )(x, wext, wdst)


def _tc_epilogue_project(acc, rep, bias2d, wext, wdst, n_blocks=10):
    n = acc.shape[0]
    blk = n // n_blocks
    return pl.pallas_call(
        _ep_mm_kernel,
        grid=(n_blocks,),
        in_specs=[
            pl.BlockSpec((blk, _ROWW), lambda i: (i, 0)),
            pl.BlockSpec((16, 128), lambda i: (0, 0)),
            pl.BlockSpec((1, 128), lambda i: (0, 0)),
            pl.BlockSpec((128, _ROWW), lambda i: (0, 0)),
            pl.BlockSpec((128, 16), lambda i: (0, 0)),
        ],
        out_specs=[
            pl.BlockSpec((blk, _ROWW), lambda i: (i, 0)),
            pl.BlockSpec((blk, 16), lambda i: (i, 0)),
        ],
        out_shape=[
            jax.ShapeDtypeStruct((n, _ROWW), jnp.float32),
            jax.ShapeDtypeStruct((n, 16), jnp.float32),
        ],
    )(acc, rep, bias2d, wext, wdst)


def _tc_epilogue_final(acc, rep, bias2d, n_blocks=10):
    n = acc.shape[0]
    blk = n // n_blocks
    return pl.pallas_call(
        _ep_final_kernel,
        grid=(n_blocks,),
        in_specs=[
            pl.BlockSpec((blk, _ROWW), lambda i: (i, 0)),
            pl.BlockSpec((16, 128), lambda i: (0, 0)),
            pl.BlockSpec((1, 128), lambda i: (0, 0)),
        ],
        out_specs=pl.BlockSpec((blk, 128), lambda i: (i, 0)),
        out_shape=jax.ShapeDtypeStruct((n, 128), jnp.float32),
    )(acc, rep, bias2d)


@jax.jit
def _sc_bucket_stage(src, dst):
    e = src.shape[0]
    ept = e // _NW
    assert ept % _L == 0

    @functools.partial(
        pl.kernel,
        out_type=[
            jax.ShapeDtypeStruct((_NW * _NW * _REG,), jnp.int32),
            jax.ShapeDtypeStruct((_NW * _NW * _REG,), jnp.int32),
            jax.ShapeDtypeStruct((_NW * _NW * 16,), jnp.int32),
        ],
        mesh=plsc.VectorSubcoreMesh(**_MESH),
        compiler_params=_SC_PARAMS,
        scratch_types=[
            pltpu.VMEM((ept,), jnp.int32),
            pltpu.VMEM((ept,), jnp.int32),
            pltpu.VMEM((_NW * _REG,), jnp.int32),
            pltpu.VMEM((_NW * _REG,), jnp.int32),
            pltpu.VMEM((_NW * 16,), jnp.int32),
        ],
    )
    def k(src_h, dst_h, ssrc_h, sdst_h, cnt_h, ebs, ebd, sts, std, cnt):
        cid = lax.axis_index("c")
        sid = lax.axis_index("s")
        wid = sid * _NC + cid
        ei = lax.iota(jnp.int32, _L)

        pltpu.sync_copy(src_h.at[pl.ds(wid * ept, ept)], ebs)
        pltpu.sync_copy(dst_h.at[pl.ds(wid * ept, ept)], ebd)

        def zc(i, _):
            cnt[pl.ds(i * _L, _L)] = jnp.zeros((_L,), jnp.int32)
            return 0
        lax.fori_loop(0, _NW, zc, 0)

        def grp(g, _):
            sv = ebs[pl.ds(g * _L, _L)]
            dv = ebd[pl.ds(g * _L, _L)]
            bv = (dv * _DIVM) >> _DIVS
            cidx = bv * _L + ei
            c = plsc.load_gather(cnt, [cidx])
            plsc.store_scatter(cnt, [cidx], c + 1)
            slot = jnp.minimum(c, _CAP16 - 1)
            sidx = bv * _REG + slot * _L + ei
            plsc.store_scatter(sts, [sidx], sv)
            plsc.store_scatter(std, [sidx], dv)
            return 0
        lax.fori_loop(0, ept // _L, grp, 0)

        pltpu.sync_copy(sts, ssrc_h.at[pl.ds(wid * _NW * _REG, _NW * _REG)])
        pltpu.sync_copy(std, sdst_h.at[pl.ds(wid * _NW * _REG, _NW * _REG)])
        pltpu.sync_copy(cnt, cnt_h.at[pl.ds(wid * _NW * 16, _NW * 16)])

    return k(src, dst)


@functools.partial(jax.jit, static_argnames=("n_nodes",))
def _sc_compact(ssrc, sdst, counts, *, n_nodes):

    @functools.partial(
        pl.kernel,
        out_type=[
            jax.ShapeDtypeStruct((_NW * _CAPB,), jnp.int32),
            jax.ShapeDtypeStruct((_NW * _CAPB,), jnp.int32),
            jax.ShapeDtypeStruct((_NW * 16,), jnp.int32),
        ],
        mesh=plsc.VectorSubcoreMesh(**_MESH),
        compiler_params=_SC_PARAMS,
        scratch_types=[
            pltpu.VMEM((_NW * _REG,), jnp.int32),
            pltpu.VMEM((_NW * _REG,), jnp.int32),
            pltpu.VMEM((_NW * 16,), jnp.int32),
            pltpu.VMEM((_CAPB,), jnp.int32),
            pltpu.VMEM((_CAPB,), jnp.int32),
            pltpu.VMEM((_L,), jnp.int32),
            pltpu.SemaphoreType.DMA,
        ],
    )
    def k(ssrc_h, sdst_h, cnt_h, dsrc_h, ddst_h, ntot_h,
          rs, rd, rc, ds_v, dd_v, nt_v, sem):
        cid = lax.axis_index("c")
        sid = lax.axis_index("s")
        wid = sid * _NC + cid
        ei = lax.iota(jnp.int32, _L)

        cps = []
        for t in range(_NW):
            off = t * _NW * _REG + wid * _REG
            cps.append(pltpu.async_copy(
                ssrc_h.at[pl.ds(off, _REG)], rs.at[pl.ds(t * _REG, _REG)], sem))
            cps.append(pltpu.async_copy(
                sdst_h.at[pl.ds(off, _REG)], rd.at[pl.ds(t * _REG, _REG)], sem))
            coff = t * _NW * 16 + wid * 16
            cps.append(pltpu.async_copy(
                cnt_h.at[pl.ds(coff, 16)], rc.at[pl.ds(t * 16, 16)], sem))

        base_fill = jnp.broadcast_to(wid * _NPB, (_L,)).astype(jnp.int32)

        def zd(i, _):
            ds_v[pl.ds(i * _L, _L)] = jnp.zeros((_L,), jnp.int32)
            dd_v[pl.ds(i * _L, _L)] = base_fill
            return 0
        lax.fori_loop(0, _CAPB // _L, zd, 0)
        for cp in cps:
            cp.wait()

        def region(t, cur):
            cvec = jnp.minimum(rc[pl.ds(t * 16, _L)], _CAP16)

            def slot(s, cur2):
                cur2 = jnp.minimum(cur2, _CAPB - _L)
                msk = cvec > s
                base = t * _REG + s * _L
                plsc.store_compressed(ds_v.at[pl.ds(cur2, _L)],
                                      rs[pl.ds(base, _L)], mask=msk)
                plsc.store_compressed(dd_v.at[pl.ds(cur2, _L)],
                                      rd[pl.ds(base, _L)], mask=msk)
                pc = plsc.all_reduce_population_count(msk)
                return cur2 + pc[0]
            return lax.fori_loop(0, _CAP16, slot, cur)
        cursor = lax.fori_loop(0, _NW, region, jnp.int32(0))

        nb = jnp.minimum(n_nodes - wid * _NPB, _NPB)
        for s in range((_NPB + _L - 1) // _L):
            lanes = s * _L + ei
            msk = lanes < nb
            vec = wid * _NPB + lanes
            cursor = jnp.minimum(cursor, _CAPB - _L)
            plsc.store_compressed(ds_v.at[pl.ds(cursor, _L)], vec, mask=msk)
            plsc.store_compressed(dd_v.at[pl.ds(cursor, _L)], vec, mask=msk)
            pc = plsc.all_reduce_population_count(msk)
            cursor = cursor + pc[0]

        nt_v[pl.ds(0, _L)] = jnp.broadcast_to(cursor, (_L,))
        pltpu.sync_copy(ds_v, dsrc_h.at[pl.ds(wid * _CAPB, _CAPB)])
        pltpu.sync_copy(dd_v, ddst_h.at[pl.ds(wid * _CAPB, _CAPB)])
        pltpu.sync_copy(nt_v, ntot_h.at[pl.ds(wid * _L, _L)])

    return k(ssrc, sdst, counts)


_NBUF = 2
_NIDX = 3
_UNROLL = 6


@functools.partial(jax.jit, static_argnames=("heads",))
def _sc_edge_pass(hext, adst_tab, dsrc, ddst, ntot, *, heads):
    out_ch = 128 // heads

    @functools.partial(
        pl.kernel,
        out_type=jax.ShapeDtypeStruct((_NW * _NPB * _ROWW,), jnp.float32),
        mesh=plsc.VectorSubcoreMesh(**_MESH),
        compiler_params=_SC_PARAMS,
        scratch_types=[
            [pltpu.VMEM((_CH,), jnp.int32)] * _NIDX,
            [pltpu.VMEM((_CH,), jnp.int32)] * _NIDX,
            [pltpu.VMEM((_CH, _ROWW), jnp.float32)] * _NBUF,
            [pltpu.VMEM((_CH, 16), jnp.float32)] * _NBUF,
            pltpu.VMEM((_NPB * _ROWW,), jnp.float32),
            pltpu.VMEM((_L,), jnp.int32),
            [pltpu.SemaphoreType.DMA] * _NBUF,
            [pltpu.SemaphoreType.DMA] * _NIDX,
        ],
    )
    def k(hext_h, adst_h, dsrc_h, ddst_h, ntot_h, out_h,
          srcidx, gdstidx, rows_v, adst_v, acc, nsm, gsem, isem):
        cid = lax.axis_index("c")
        sid = lax.axis_index("s")
        wid = sid * _NC + cid
        ei = lax.iota(jnp.int32, _L)
        ebase = wid * _CAPB

        pltpu.sync_copy(ntot_h.at[pl.ds(wid * _L, _L)], nsm)
        n_real = nsm[pl.ds(0, _L)][0]
        nch = (n_real + _CH - 1) // _CH

        def zacc(i, _):
            acc[pl.ds(i * _L, _L)] = jnp.zeros((_L,), jnp.float32)
            return 0
        lax.fori_loop(0, _NPB * _ROWW // _L, zacc, 0)

        def issue_idx(g, q):
            pltpu.async_copy(dsrc_h.at[pl.ds(ebase + g * _CH, _CH)],
                             srcidx[q], isem[q])
            pltpu.async_copy(ddst_h.at[pl.ds(ebase + g * _CH, _CH)],
                             gdstidx[q], isem[q])

        def wait_idx(q):
            pltpu.make_async_copy(dsrc_h.at[pl.ds(0, _CH)], srcidx[q], isem[q]).wait()
            pltpu.make_async_copy(dsrc_h.at[pl.ds(0, _CH)], gdstidx[q], isem[q]).wait()

        def issue_gather(b, q):
            pltpu.async_copy(hext_h.at[srcidx[q]], rows_v[b], gsem[b])
            pltpu.async_copy(adst_h.at[gdstidx[q]], adst_v[b], gsem[b])

        def wait_gather(b):
            pltpu.make_async_copy(hext_h.at[pl.ds(0, _CH)], rows_v[b], gsem[b]).wait()
            pltpu.make_async_copy(adst_h.at[pl.ds(0, _CH)], adst_v[b], gsem[b]).wait()

        def compute(g, b, q):
            base = g * _CH
            rv = rows_v[b]
            av = adst_v[b]
            gq = gdstidx[q]

            def escale(p, _):
                ws = []
                idxs = []
                for ee in range(8):
                    e = 8 * p + ee
                    efull = jnp.full((_L,), e, jnp.int32)
                    a_s = rv[e, pl.ds(128, _L)]
                    a_d = av[e, pl.ds(0, _L)]
                    t = a_s + a_d
                    t = jnp.where(t >= 0, t, 0.2 * t)
                    valid = (base + e) < n_real
                    w16 = jnp.where((ei < heads) & valid, jnp.exp(t), 0.0)
                    ldb = plsc.load_gather(gq, [efull]) - wid * _NPB
                    ws.append(w16)
                    idxs.append(ldb * _ROWW + ei)
                for j in range(8):
                    hj = (j * 16) // out_ch
                    hjf = jnp.full((_L,), hj, jnp.int32)
                    for ee in range(8):
                        e = 8 * p + ee
                        wb = ws[ee].at[hjf].get(mode="promise_in_bounds")
                        plsc.addupdate_scatter(
                            acc, [idxs[ee] + j * 16],
                            rv[e, pl.ds(j * 16, 16)] * wb)
                for ee in range(8):
                    plsc.addupdate_scatter(acc, [idxs[ee] + 128], ws[ee])
                return 0
            lax.fori_loop(0, _CH // 8, escale, 0)

        issue_idx(0, 0)
        issue_idx(1, 1)
        wait_idx(0)
        issue_gather(0, 0)

        def trip(t, _):
            for kk in range(_UNROLL):
                g = _UNROLL * t + kk
                b = kk % _NBUF
                bn = (kk + 1) % _NBUF
                qn = (kk + 1) % _NIDX
                qnn = (kk + 2) % _NIDX

                @pl.when(g < nch)
                def _():
                    @pl.when(g + 1 < nch)
                    def _():
                        wait_idx(qn)
                        issue_gather(bn, qn)

                    @pl.when(g + 2 < nch)
                    def _():
                        issue_idx(g + 2, qnn)
                    wait_gather(b)
                    compute(g, b, kk % _NIDX)
            return 0
        lax.fori_loop(0, (nch + _UNROLL - 1) // _UNROLL, trip, 0)

        pltpu.sync_copy(acc, out_h.at[pl.ds(wid * _NPB * _ROWW, _NPB * _ROWW)])

    return k(hext, adst_tab, dsrc, ddst, ntot).reshape(_NW * _NPB, _ROWW)


def _fold_weights(W, att_src, att_dst, heads, out_ch):
    w3 = W.reshape(W.shape[0], heads, out_ch)
    wsrc = jnp.sum(w3 * att_src, axis=-1)
    wdst = jnp.sum(w3 * att_dst, axis=-1)
    pad = jnp.zeros((W.shape[0], 16 - heads), jnp.float32)
    wext = jnp.concatenate([W, wsrc, pad], axis=1)
    wdst16 = jnp.concatenate([wdst, pad], axis=1)
    return wext, wdst16


def _rep_matrix(heads):
    out_ch = 128 // heads
    rep = jnp.zeros((16, 128), jnp.float32)
    hc = jnp.arange(128) // out_ch
    rep = rep.at[hc, jnp.arange(128)].set(1.0)
    return rep


def kernel(inputs, edge_index, W1, att_src1, att_dst1, bias1,
           W2, att_src2, att_dst2, bias2):
    N, D = inputs.shape
    E = edge_index.shape[1]
    assert E % (_NW * _L) == 0 and _NW * _NPB >= N

    wext1, wdst1 = _fold_weights(W1, att_src1, att_dst1, 8, 16)
    wext2, wdst2 = _fold_weights(W2, att_src2, att_dst2, 1, 128)
    rep1 = _rep_matrix(8)
    rep2 = _rep_matrix(1)
    b1 = bias1.reshape(1, 128)
    b2 = bias2.reshape(1, 128)

    ssrc, sdst, counts = _sc_bucket_stage(edge_index[0], edge_index[1])
    dsrc, ddst, ntot = _sc_compact(ssrc, sdst, counts, n_nodes=N)

    hext1, adst1 = _tc_project(inputs, wext1, wdst1)
    acc1 = _sc_edge_pass(hext1, adst1, dsrc, ddst, ntot, heads=8)
    hext2, adst2 = _tc_epilogue_project(acc1[:N], rep1, b1, wext2, wdst2)
    acc2 = _sc_edge_pass(hext2, adst2, dsrc, ddst, ntot, heads=1)
    return _tc_epilogue_final(acc2[:N], rep2, b2)
